# baseline (speedup 1.0000x reference)
import functools

import jax
import jax.numpy as jnp
from jax import lax
from jax.experimental import pallas as pl
from jax.experimental.pallas import tpu as pltpu
from jax.experimental.pallas import tpu_sc as plsc

_B = 64
_K1 = 4097
_D = 128
_N = 100000
_T = 0.07
_KP = 4112
_NPACK_H = 64
_NTILES = 32
_TASKS_PER_TILE = _NPACK_H // _NTILES
_CHUNK = 8192
_NBLK = 13
_NP = _NBLK * _CHUNK


def _packed(lhs, rhs):
    dn = (((1,), (1,)), ((), ()))
    prod = lax.dot_general(lhs, rhs, dn, preferred_element_type=jnp.float32)
    return pltpu.bitcast(prod.astype(jnp.bfloat16), jnp.int32)


def _tc_body(v_ref, m_ref, out_ref):
    nrow = out_ref.shape[0]
    out_ref[...] = _packed(v_ref[...], m_ref[...]).reshape(
        nrow, _CHUNK // 128, 128)


def _tc_call(v, bank):
    npack = v.shape[0] // 2
    full = lambda shape: pl.BlockSpec(shape, lambda i: (0, 0))
    chunk = pl.BlockSpec((_CHUNK, _D), lambda i: (i, 0))
    return pl.pallas_call(
        _tc_body,
        grid=(_NBLK,),
        in_specs=[full(v.shape), chunk],
        out_specs=pl.BlockSpec((npack, _CHUNK // 128, 128),
                               lambda i: (0, i, 0)),
        out_shape=jax.ShapeDtypeStruct((npack, _NP // 128, 128), jnp.int32),
    )(v, bank)


_sc_mesh = plsc.VectorSubcoreMesh(core_axis_name="c", subcore_axis_name="s")


def _make_sc_gather(npack):
    tasks_per_tile = -(-npack // _NTILES)

    @functools.partial(
        pl.kernel,
        out_type=jax.ShapeDtypeStruct((2 * npack * _KP,), jnp.float32),
        mesh=_sc_mesh,
        compiler_params=pltpu.CompilerParams(needs_layout_passes=False),
        scratch_types=[
            pltpu.VMEM((_NP,), jnp.int32),
            pltpu.VMEM((_KP,), jnp.int32),
            pltpu.VMEM((_KP,), jnp.int32),
            pltpu.VMEM((_KP,), jnp.float32),
            pltpu.VMEM((_KP,), jnp.float32),
            pltpu.SemaphoreType.DMA,
        ],
    )
    def _sc_gather(p_hbm, idx_hbm, out_hbm, prow_v, idx0_v, idx1_v,
                   out0_v, out1_v, sem):
        wid = lax.axis_index("s") * 2 + lax.axis_index("c")
        inv_t = jnp.float32(1.0 / _T)
        hi_mask = jnp.int32(-65536)

        def task(t, carry):
            m = wid * tasks_per_tile + t
            r0 = 2 * m
            b0 = lax.rem(r0, _B)
            b1 = lax.rem(r0 + 1, _B)
            cp_p = pltpu.async_copy(
                p_hbm.at[pl.ds(pl.multiple_of(m * _NP, 8), _NP)], prow_v, sem)
            cp_i0 = pltpu.async_copy(
                idx_hbm.at[pl.ds(pl.multiple_of(b0 * _KP, 8), _KP)], idx0_v,
                sem)
            cp_i1 = pltpu.async_copy(
                idx_hbm.at[pl.ds(pl.multiple_of(b1 * _KP, 8), _KP)], idx1_v,
                sem)
            cp_i0.wait()
            cp_i1.wait()
            cp_p.wait()

            def col(i, c):
                sl = pl.ds(i * 16, 16)
                w0 = plsc.load_gather(prow_v, [idx0_v[sl]])
                out0_v[sl] = plsc.bitcast(
                    lax.shift_left(w0, 16), jnp.float32) * inv_t
                w1 = plsc.load_gather(prow_v, [idx1_v[sl]])
                out1_v[sl] = plsc.bitcast(
                    lax.bitwise_and(w1, hi_mask), jnp.float32) * inv_t
                return c

            lax.fori_loop(0, _KP // 16, col, 0, unroll=4)
            pltpu.sync_copy(
                out0_v, out_hbm.at[pl.ds(pl.multiple_of(r0 * _KP, 8), _KP)])
            pltpu.sync_copy(
                out1_v,
                out_hbm.at[pl.ds(pl.multiple_of((r0 + 1) * _KP, 8), _KP)])
            return carry

        lax.fori_loop(0, tasks_per_tile, task, 0)

    return _sc_gather


_sc_gather64 = _make_sc_gather(64)
_sc_gather32 = _make_sc_gather(32)


def kernel(v1, v2, v3, y, idx, memory_v1, memory_v2, memory_v3):
    del y
    v12 = jnp.concatenate([v1, v2], axis=0)
    pa = _tc_call(v12, memory_v3)
    pb1 = _tc_call(v3, memory_v1)
    pb2 = _tc_call(v3, memory_v2)
    idx_p = jnp.pad(idx, ((0, 0), (0, _KP - _K1))).reshape(-1)
    fa = _sc_gather64(pa.reshape(-1), idx_p)
    fb1 = _sc_gather32(pb1.reshape(-1), idx_p)
    fb2 = _sc_gather32(pb2.reshape(-1), idx_p)
    oa = fa.reshape(2, _B, _KP)[:, :, :_K1, None]
    return (oa[0], oa[1],
            fb1.reshape(_B, _KP)[:, :_K1, None],
            fb2.reshape(_B, _KP)[:, :_K1, None])


# speedup vs baseline: 1.0271x; 1.0271x over previous
import functools

import jax
import jax.numpy as jnp
from jax import lax
from jax.experimental import pallas as pl
from jax.experimental.pallas import tpu as pltpu
from jax.experimental.pallas import tpu_sc as plsc

_B = 64
_K1 = 4097
_D = 128
_N = 100000
_T = 0.07
_KP = 4112
_NPACK_H = 64
_NTILES = 32
_TASKS_PER_TILE = _NPACK_H // _NTILES
_CHUNK = 8192
_NBLK = 13
_NP = _NBLK * _CHUNK


def _packed(lhs, rhs):
    dn = (((1,), (1,)), ((), ()))
    prod = lax.dot_general(lhs, rhs, dn, preferred_element_type=jnp.float32)
    return pltpu.bitcast(prod.astype(jnp.bfloat16), jnp.int32)


def _tc_body(v_ref, m_ref, out_ref):
    nrow = out_ref.shape[0]
    out_ref[...] = _packed(v_ref[...], m_ref[...]).reshape(
        nrow, _CHUNK // 128, 128)


def _tc_call(v, bank):
    npack = v.shape[0] // 2
    full = lambda shape: pl.BlockSpec(shape, lambda i: (0, 0))
    chunk = pl.BlockSpec((_CHUNK, _D), lambda i: (i, 0))
    return pl.pallas_call(
        _tc_body,
        grid=(_NBLK,),
        in_specs=[full(v.shape), chunk],
        out_specs=pl.BlockSpec((npack, _CHUNK // 128, 128),
                               lambda i: (0, i, 0)),
        out_shape=jax.ShapeDtypeStruct((npack, _NP // 128, 128), jnp.int32),
    )(v, bank)


_sc_mesh = plsc.VectorSubcoreMesh(core_axis_name="c", subcore_axis_name="s")


def _make_sc_gather(npack):
    tasks_per_tile = -(-npack // _NTILES)

    @functools.partial(
        pl.kernel,
        out_type=jax.ShapeDtypeStruct((2 * npack * _KP,), jnp.float32),
        mesh=_sc_mesh,
        compiler_params=pltpu.CompilerParams(needs_layout_passes=False),
        scratch_types=[
            pltpu.VMEM((_NP,), jnp.int32),
            pltpu.VMEM((_KP,), jnp.int32),
            pltpu.VMEM((_KP,), jnp.int32),
            pltpu.VMEM((_KP,), jnp.float32),
            pltpu.VMEM((_KP,), jnp.float32),
            pltpu.SemaphoreType.DMA,
        ],
    )
    def _sc_gather(p_hbm, idx_hbm, out_hbm, prow_v, idx0_v, idx1_v,
                   out0_v, out1_v, sem):
        wid = lax.axis_index("s") * 2 + lax.axis_index("c")
        inv_t = jnp.float32(1.0 / _T)
        hi_mask = jnp.int32(-65536)

        def task(t, carry):
            m = wid * tasks_per_tile + t
            r0 = 2 * m
            b0 = lax.rem(r0, _B)
            b1 = lax.rem(r0 + 1, _B)
            cp_p = pltpu.async_copy(
                p_hbm.at[pl.ds(pl.multiple_of(m * _NP, 8), _NP)], prow_v, sem)
            cp_i0 = pltpu.async_copy(
                idx_hbm.at[pl.ds(pl.multiple_of(b0 * _KP, 8), _KP)], idx0_v,
                sem)
            cp_i1 = pltpu.async_copy(
                idx_hbm.at[pl.ds(pl.multiple_of(b1 * _KP, 8), _KP)], idx1_v,
                sem)
            cp_i0.wait()
            cp_i1.wait()
            cp_p.wait()

            def col(i, c):
                sl = pl.ds(i * 16, 16)
                w0 = plsc.load_gather(prow_v, [idx0_v[sl]])
                out0_v[sl] = plsc.bitcast(
                    lax.shift_left(w0, 16), jnp.float32) * inv_t
                w1 = plsc.load_gather(prow_v, [idx1_v[sl]])
                out1_v[sl] = plsc.bitcast(
                    lax.bitwise_and(w1, hi_mask), jnp.float32) * inv_t
                return c

            lax.fori_loop(0, _KP // 16, col, 0)
            pltpu.sync_copy(
                out0_v, out_hbm.at[pl.ds(pl.multiple_of(r0 * _KP, 8), _KP)])
            pltpu.sync_copy(
                out1_v,
                out_hbm.at[pl.ds(pl.multiple_of((r0 + 1) * _KP, 8), _KP)])
            return carry

        lax.fori_loop(0, tasks_per_tile, task, 0)

    return _sc_gather


_sc_gather64 = _make_sc_gather(64)
_sc_gather32 = _make_sc_gather(32)


def kernel(v1, v2, v3, y, idx, memory_v1, memory_v2, memory_v3):
    del y
    v12 = jnp.concatenate([v1, v2], axis=0)
    pa = _tc_call(v12, memory_v3)
    pb1 = _tc_call(v3, memory_v1)
    pb2 = _tc_call(v3, memory_v2)
    idx_p = jnp.pad(idx, ((0, 0), (0, _KP - _K1))).reshape(-1)
    fa = _sc_gather64(pa.reshape(-1), idx_p)
    fb1 = _sc_gather32(pb1.reshape(-1), idx_p)
    fb2 = _sc_gather32(pb2.reshape(-1), idx_p)
    oa = fa.reshape(2, _B, _KP)[:, :, :_K1, None]
    return (oa[0], oa[1],
            fb1.reshape(_B, _KP)[:, :_K1, None],
            fb2.reshape(_B, _KP)[:, :_K1, None])
